# baseline (speedup 1.0000x reference)
import functools

import numpy as np
import jax
import jax.numpy as jnp
from jax import lax
from jax.experimental import pallas as pl
from jax.experimental.pallas import tpu as pltpu
from jax.experimental.pallas import tpu_sc as plsc

B, N, D = 32, 4096, 1024
KEEP = N // 2
ROWS = B * KEEP

NC, NS = 2, 16
NW = NC * NS
PER_W = ROWS // NW
CHUNK = 32
NCHUNK = PER_W // CHUNK


_R0 = (13, 15, 26, 6)
_R1 = (17, 29, 16, 24)


def _threefry2x32(k0, k1, x0, x1):
    x0 = np.asarray(x0, np.uint32).copy()
    x1 = np.asarray(x1, np.uint32).copy()
    ks0, ks1 = np.uint32(k0), np.uint32(k1)
    ks2 = np.uint32(ks0 ^ ks1 ^ np.uint32(0x1BD11BDA))
    with np.errstate(over="ignore"):
        x0 = (x0 + ks0).astype(np.uint32)
        x1 = (x1 + ks1).astype(np.uint32)
        sched = [(ks1, ks2), (ks2, ks0), (ks0, ks1), (ks1, ks2), (ks2, ks0)]
        for r in range(5):
            for rot in (_R0 if r % 2 == 0 else _R1):
                x0 = (x0 + x1).astype(np.uint32)
                x1 = ((x1 << np.uint32(rot)) |
                      (x1 >> np.uint32(32 - rot))).astype(np.uint32)
                x1 = (x1 ^ x0).astype(np.uint32)
            a, b = sched[r]
            x0 = (x0 + a).astype(np.uint32)
            x1 = (x1 + b + np.uint32(r + 1)).astype(np.uint32)
    return x0, x1


def _bits32(k0, k1, n):
    i = np.arange(n, dtype=np.uint64)
    c1 = (i >> np.uint64(32)).astype(np.uint32)
    c2 = (i & np.uint64(0xFFFFFFFF)).astype(np.uint32)
    b1, b2 = _threefry2x32(k0, k1, c1, c2)
    return b1 ^ b2


def _split2(k0, k1):
    i = np.arange(2, dtype=np.uint64)
    c1 = (i >> np.uint64(32)).astype(np.uint32)
    c2 = (i & np.uint64(0xFFFFFFFF)).astype(np.uint32)
    b1, b2 = _threefry2x32(k0, k1, c1, c2)
    return (b1[0], b2[0]), (b1[1], b2[1])


def _np_permutation(seed, n):
    key = (np.uint32(seed >> 32), np.uint32(seed & 0xFFFFFFFF))
    x = np.arange(n, dtype=np.int64)
    num_rounds = int(np.ceil(3 * np.log(max(1, n)) /
                             np.log(np.iinfo(np.uint32).max)))
    for _ in range(num_rounds):
        key, sub = _split2(*key)
        sort_keys = _bits32(sub[0], sub[1], n)
        x = x[np.argsort(sort_keys, kind="stable")]
    return x


def _flat_idx() -> np.ndarray:
    perm = _np_permutation(42, N)[:KEEP]
    flat = np.arange(B, dtype=np.int64)[:, None] * N + perm[None, :]
    return np.ascontiguousarray(flat.reshape(-1).astype(np.int32))


_IDX = _flat_idx()
_PERM = np.ascontiguousarray(_np_permutation(42, N)[:KEEP].astype(np.int32))


_mesh = plsc.VectorSubcoreMesh(core_axis_name="c", subcore_axis_name="s")


@functools.partial(
    pl.kernel,
    mesh=_mesh,
    out_type=jax.ShapeDtypeStruct((ROWS, D), jnp.float32),
    scratch_types=[
        pltpu.VMEM((PER_W,), jnp.int32),
        pltpu.SemaphoreType.DMA,
    ],
)
def _gather_rows(x_hbm, idx_hbm, out_hbm, idx_v, sem):
    wid = lax.axis_index("s") * NC + lax.axis_index("c")
    base = wid * PER_W
    pltpu.sync_copy(idx_hbm.at[pl.ds(base, PER_W)], idx_v)

    def _copy(c):
        off = pl.multiple_of(c * CHUNK, 8)
        return pltpu.make_async_copy(
            x_hbm.at[idx_v.at[pl.ds(off, CHUNK)]],
            out_hbm.at[pl.ds(base + off, CHUNK)], sem)

    DEPTH = 4

    for c in range(DEPTH):
        _copy(c).start()

    def body(j, carry):
        _copy(j).wait()
        _copy(j + DEPTH).start()
        return carry

    lax.fori_loop(0, NCHUNK - DEPTH, body, 0)
    for c in range(NCHUNK - DEPTH, NCHUNK):
        _copy(c).wait()


_TC_DEPTH = 8


def _tc_body(idx_ref, x_ref, out_ref, sem):
    n = out_ref.shape[1]

    def _cp(i):
        r = idx_ref[i]
        return pltpu.make_async_copy(
            x_ref.at[:, pl.ds(r, 1), :], out_ref.at[:, pl.ds(i, 1), :], sem)

    for i in range(_TC_DEPTH):
        _cp(i).start()

    def body(i, carry):
        _cp(i).wait()
        _cp(i + _TC_DEPTH).start()
        return carry

    lax.fori_loop(0, n - _TC_DEPTH, body, 0)
    for i in range(n - _TC_DEPTH, n):
        _cp(i).wait()


def _tc_gather(x, perm_idx):
    n = perm_idx.shape[0]
    return pl.pallas_call(
        _tc_body,
        out_shape=jax.ShapeDtypeStruct((B, n, D), jnp.float32),
        in_specs=[
            pl.BlockSpec(memory_space=pltpu.SMEM),
            pl.BlockSpec(memory_space=pl.ANY),
        ],
        out_specs=pl.BlockSpec(memory_space=pl.ANY),
        scratch_shapes=[pltpu.SemaphoreType.DMA],
    )(perm_idx, x)


def kernel(x):
    idx = jnp.asarray(_PERM)
    return _tc_gather(x, idx)


# speedup vs baseline: 39.6218x; 39.6218x over previous
import functools

import numpy as np
import jax
import jax.numpy as jnp
from jax import lax
from jax.experimental import pallas as pl
from jax.experimental.pallas import tpu as pltpu
from jax.experimental.pallas import tpu_sc as plsc

B, N, D = 32, 4096, 1024
KEEP = N // 2
ROWS = B * KEEP

NC, NS = 2, 16
NW = NC * NS
PER_W = ROWS // NW
CHUNK = 16
RING = 4
NCHUNK = PER_W // CHUNK


_R0 = (13, 15, 26, 6)
_R1 = (17, 29, 16, 24)


def _threefry2x32(k0, k1, x0, x1):
    x0 = np.asarray(x0, np.uint32).copy()
    x1 = np.asarray(x1, np.uint32).copy()
    ks0, ks1 = np.uint32(k0), np.uint32(k1)
    ks2 = np.uint32(ks0 ^ ks1 ^ np.uint32(0x1BD11BDA))
    with np.errstate(over="ignore"):
        x0 = (x0 + ks0).astype(np.uint32)
        x1 = (x1 + ks1).astype(np.uint32)
        sched = [(ks1, ks2), (ks2, ks0), (ks0, ks1), (ks1, ks2), (ks2, ks0)]
        for r in range(5):
            for rot in (_R0 if r % 2 == 0 else _R1):
                x0 = (x0 + x1).astype(np.uint32)
                x1 = ((x1 << np.uint32(rot)) |
                      (x1 >> np.uint32(32 - rot))).astype(np.uint32)
                x1 = (x1 ^ x0).astype(np.uint32)
            a, b = sched[r]
            x0 = (x0 + a).astype(np.uint32)
            x1 = (x1 + b + np.uint32(r + 1)).astype(np.uint32)
    return x0, x1


def _bits32(k0, k1, n):
    i = np.arange(n, dtype=np.uint64)
    c1 = (i >> np.uint64(32)).astype(np.uint32)
    c2 = (i & np.uint64(0xFFFFFFFF)).astype(np.uint32)
    b1, b2 = _threefry2x32(k0, k1, c1, c2)
    return b1 ^ b2


def _split2(k0, k1):
    i = np.arange(2, dtype=np.uint64)
    c1 = (i >> np.uint64(32)).astype(np.uint32)
    c2 = (i & np.uint64(0xFFFFFFFF)).astype(np.uint32)
    b1, b2 = _threefry2x32(k0, k1, c1, c2)
    return (b1[0], b2[0]), (b1[1], b2[1])


def _np_permutation(seed, n):
    key = (np.uint32(seed >> 32), np.uint32(seed & 0xFFFFFFFF))
    x = np.arange(n, dtype=np.int64)
    num_rounds = int(np.ceil(3 * np.log(max(1, n)) /
                             np.log(np.iinfo(np.uint32).max)))
    for _ in range(num_rounds):
        key, sub = _split2(*key)
        sort_keys = _bits32(sub[0], sub[1], n)
        x = x[np.argsort(sort_keys, kind="stable")]
    return x


def _flat_idx() -> np.ndarray:
    perm = _np_permutation(42, N)[:KEEP]
    flat = np.arange(B, dtype=np.int64)[:, None] * N + perm[None, :]
    return np.ascontiguousarray(flat.reshape(-1).astype(np.int32))


_IDX = _flat_idx()


_mesh = plsc.VectorSubcoreMesh(core_axis_name="c", subcore_axis_name="s")


@functools.partial(
    pl.kernel,
    mesh=_mesh,
    out_type=jax.ShapeDtypeStruct((ROWS, D), jnp.float32),
    scratch_types=(
        [pltpu.VMEM((PER_W,), jnp.int32)]
        + [pltpu.VMEM((CHUNK, D), jnp.float32) for _ in range(RING)]
        + [pltpu.SemaphoreType.DMA for _ in range(2 * RING)]
    ),
)
def _gather_rows(x_hbm, idx_hbm, out_hbm, idx_v, *bufs_sems):
    buf = bufs_sems[:RING]
    gs = bufs_sems[RING:2 * RING]
    ss = bufs_sems[2 * RING:]

    wid = lax.axis_index("s") * NC + lax.axis_index("c")
    base = wid * PER_W
    pltpu.sync_copy(idx_hbm.at[pl.ds(base, PER_W)], idx_v)

    def _gather(c, k):
        off = pl.multiple_of(c * CHUNK, 8)
        return pltpu.make_async_copy(
            x_hbm.at[idx_v.at[pl.ds(off, CHUNK)]], buf[k], gs[k])

    def _scatter(c, k):
        off = pl.multiple_of(base + c * CHUNK, 8)
        return pltpu.make_async_copy(buf[k], out_hbm.at[pl.ds(off, CHUNK)], ss[k])

    for c in range(RING - 1):
        _gather(c, c).start()

    _gather(0, 0).wait()
    _scatter(0, 0).start()
    _gather(RING - 1, RING - 1).start()

    def body(j, carry):
        for u in range(RING):
            c = RING * j + 1 + u
            k = (1 + u) % RING
            _gather(c, k).wait()
            _scatter(c, k).start()
            _scatter(c - 1, u % RING).wait()
            _gather(c + RING - 1, u % RING).start()
        return carry

    lax.fori_loop(0, (NCHUNK - RING) // RING, body, 0)

    for c in range(NCHUNK - RING + 1, NCHUNK):
        k = c % RING
        _gather(c, k).wait()
        _scatter(c, k).start()
        _scatter(c - 1, (c - 1) % RING).wait()
    _scatter(NCHUNK - 1, (NCHUNK - 1) % RING).wait()


def kernel(x):
    idx = jnp.asarray(_IDX)
    out = _gather_rows(x.reshape(B * N, D), idx)
    return out.reshape(B, KEEP, D)
